# baseline (speedup 1.0000x reference)
import functools

import jax
import jax.numpy as jnp
from jax import lax
from jax.experimental import pallas as pl
from jax.experimental.pallas import tpu as pltpu
from jax.experimental.pallas import tpu_sc as plsc

N = 10000
E = 320000
D_IN = 128
H1 = 32
H2 = 16
D_OUT = 2

NC = 2
NS = 16
NW = NC * NS

W = 128
STEPS = 80
NBUF = 4
GROUPS = STEPS // NBUF
EPW = W * STEPS
E_PAD = EPW * NW
N_PAD = 10112
RPT = N_PAD // NS
N_SPARE = N_PAD - N

_f32 = jnp.float32

_SC_PARAMS = pltpu.CompilerParams(use_tc_tiling_on_sc=False)


def _mesh():
    return plsc.VectorSubcoreMesh(core_axis_name="c", subcore_axis_name="s")


def _hist_body(dst_hbm, out_hbm, acc_sp, didx, ones, zeros, vbuf, sem0, sem1,
               sem2, sem3):
    c = lax.axis_index("c")
    s = lax.axis_index("s")
    wid = s * NC + c
    rbase = s * RPT
    ssem = [sem0, sem1, sem2, sem3]
    pltpu.sync_copy(dst_hbm.at[pl.ds(wid * STEPS, STEPS)], didx)
    for j in range(W // 16):
        ones[pl.ds(j * 16, 16)] = jnp.ones((16,), _f32)
        zeros[pl.ds(j * 16, 16)] = jnp.zeros((16,), _f32)
    for k in range(RPT // W):
        pltpu.sync_copy(zeros, acc_sp.at[pl.ds(rbase + k * W, W)])
    pltpu.sync_copy(zeros.at[pl.ds(0, RPT % W)],
                    acc_sp.at[pl.ds(rbase + (RPT // W) * W, RPT % W)])
    plsc.subcore_barrier()

    def group(g, carry):
        for b in range(NBUF):
            t = g * NBUF + b

            @pl.when(g > 0)
            def _():
                pltpu.make_async_copy(ones, acc_sp.at[didx.at[t]],
                                      ssem[b]).wait()

            pltpu.async_copy(ones, acc_sp.at[didx.at[t]], ssem[b], add=True)
        return carry

    lax.fori_loop(0, GROUPS, group, 0)
    for b in range(NBUF):
        pltpu.make_async_copy(ones, acc_sp.at[didx.at[0]], ssem[b]).wait()
    plsc.subcore_barrier()
    obase = pl.multiple_of(c * N_PAD + rbase, 8)
    for k in range(RPT // W + 1):
        ln = W if k < RPT // W else RPT % W
        pltpu.sync_copy(acc_sp.at[pl.ds(rbase + k * W, ln)], vbuf.at[pl.ds(0, ln)])
        pltpu.sync_copy(vbuf.at[pl.ds(0, ln)], out_hbm.at[pl.ds(obase + k * W, ln)])


_hist = functools.partial(
    pl.kernel,
    out_type=jax.ShapeDtypeStruct((NC * N_PAD,), _f32),
    mesh=_mesh(),
    compiler_params=_SC_PARAMS,
    scratch_types=[
        pltpu.VMEM_SHARED((N_PAD,), _f32),
        pltpu.VMEM((STEPS, W), jnp.int32),
        pltpu.VMEM((W,), _f32),
        pltpu.VMEM((W,), _f32),
        pltpu.VMEM((W,), _f32),
        pltpu.SemaphoreType.DMA,
        pltpu.SemaphoreType.DMA,
        pltpu.SemaphoreType.DMA,
        pltpu.SemaphoreType.DMA,
    ],
)(_hist_body)


ANBUF = 8
LOOK = ANBUF // 2
AGROUPS = STEPS // ANBUF


def _agg_body_for(D):
    def agg(xs_hbm, src_hbm, dst_hbm, out_hbm, acc_sp, sidx, didx,
            *bufs):
        c = lax.axis_index("c")
        s = lax.axis_index("s")
        wid = s * NC + c
        rbase = s * RPT
        rows = list(bufs[:ANBUF])
        gsem = list(bufs[ANBUF:2 * ANBUF])
        ssem = list(bufs[2 * ANBUF:3 * ANBUF])
        pltpu.async_copy(src_hbm.at[pl.ds(wid * STEPS, STEPS)], sidx, gsem[1])
        pltpu.async_copy(dst_hbm.at[pl.ds(wid * STEPS, STEPS)], didx, gsem[2])

        def zrow(i, carry):
            for j in range(D // 16):
                rows[0][i, pl.ds(j * 16, 16)] = jnp.zeros((16,), _f32)
            return carry

        lax.fori_loop(0, W, zrow, 0)
        pltpu.make_async_copy(src_hbm.at[pl.ds(wid * STEPS, STEPS)], sidx,
                              gsem[1]).wait()
        pltpu.make_async_copy(dst_hbm.at[pl.ds(wid * STEPS, STEPS)], didx,
                              gsem[2]).wait()
        for k in range(RPT // W):
            pltpu.sync_copy(rows[0], acc_sp.at[pl.ds(rbase + k * W, W)])
        pltpu.sync_copy(rows[0].at[pl.ds(0, RPT % W)],
                        acc_sp.at[pl.ds(rbase + (RPT // W) * W, RPT % W)])
        plsc.subcore_barrier()

        for t0 in range(LOOK):
            pltpu.async_copy(xs_hbm.at[sidx.at[t0]], rows[t0], gsem[t0])

        def group(g, carry):
            for b in range(ANBUF):
                t = g * ANBUF + b
                bp = (b + LOOK) % ANBUF
                pltpu.make_async_copy(xs_hbm.at[sidx.at[t]], rows[b],
                                      gsem[b]).wait()
                pltpu.async_copy(rows[b], acc_sp.at[didx.at[t]], ssem[b],
                                 add=True)
                if b < ANBUF - LOOK:
                    @pl.when(g > 0)
                    def _():
                        pltpu.make_async_copy(rows[bp], acc_sp.at[didx.at[t]],
                                              ssem[bp]).wait()

                    pltpu.async_copy(xs_hbm.at[sidx.at[t + LOOK]], rows[bp],
                                     gsem[bp])
                else:
                    @pl.when(g < AGROUPS - 1)
                    def _():
                        pltpu.make_async_copy(rows[bp], acc_sp.at[didx.at[t]],
                                              ssem[bp]).wait()
                        pltpu.async_copy(xs_hbm.at[sidx.at[t + LOOK]], rows[bp],
                                         gsem[bp])
            return carry

        lax.fori_loop(0, AGROUPS, group, 0)
        for b in range(ANBUF):
            pltpu.make_async_copy(rows[b], acc_sp.at[didx.at[0]],
                                  ssem[b]).wait()
        plsc.subcore_barrier()
        pltpu.sync_copy(acc_sp.at[pl.ds(rbase, RPT)],
                        out_hbm.at[c, pl.ds(rbase, RPT)])

    return agg


def _make_agg(D):
    return functools.partial(
        pl.kernel,
        out_type=jax.ShapeDtypeStruct((NC, N_PAD, D), _f32),
        mesh=_mesh(),
        compiler_params=_SC_PARAMS,
        scratch_types=[
            pltpu.VMEM_SHARED((N_PAD, D), _f32),
            pltpu.VMEM((STEPS, W), jnp.int32),
            pltpu.VMEM((STEPS, W), jnp.int32),
        ] + [pltpu.VMEM((W, D), _f32)] * ANBUF
          + [pltpu.SemaphoreType.DMA] * (2 * ANBUF),
    )(_agg_body_for(D))


_agg32 = _make_agg(H1)
_agg16 = _make_agg(H2)


def _edges_body(ei_ref, src_ref, dst_ref):
    pad = N + lax.rem(
        lax.broadcasted_iota(jnp.int32, (E_PAD - E,), 0), N_SPARE)
    src_ref[pl.ds(0, E)] = ei_ref[0]
    dst_ref[pl.ds(0, E)] = ei_ref[1]
    src_ref[pl.ds(E, E_PAD - E)] = pad
    dst_ref[pl.ds(E, E_PAD - E)] = pad


_edges = pl.pallas_call(
    _edges_body,
    out_shape=(jax.ShapeDtypeStruct((E_PAD,), jnp.int32),
               jax.ShapeDtypeStruct((E_PAD,), jnp.int32)),
)


def _prep_body(degp_ref, x_ref, w1_ref, dinv_ref, xs1_ref):
    deg = jnp.sum(degp_ref[...], axis=0, keepdims=True) + 1.0
    dinv = lax.rsqrt(deg)
    mask = (lax.broadcasted_iota(jnp.int32, (1, N_PAD), 1) < N).astype(_f32)
    dinv = dinv * mask
    dinv_ref[...] = dinv
    dinv_col = jnp.transpose(dinv)
    xs1_ref[...] = jnp.dot(x_ref[...], w1_ref[...],
                           preferred_element_type=_f32) * dinv_col


_prep = pl.pallas_call(
    _prep_body,
    out_shape=(jax.ShapeDtypeStruct((1, N_PAD), _f32),
               jax.ShapeDtypeStruct((N_PAD, H1), _f32)),
)


def _mid1_body(agg_ref, xs_ref, dinv_ref, b_ref, w_ref, out_ref):
    dinv = jnp.transpose(dinv_ref[...])
    t = agg_ref[0] + agg_ref[1] + xs_ref[...]
    h = jnp.maximum(dinv * t + b_ref[...], 0.0)
    out_ref[...] = jnp.dot(h, w_ref[...], preferred_element_type=_f32) * dinv


_mid1 = pl.pallas_call(
    _mid1_body,
    out_shape=jax.ShapeDtypeStruct((N_PAD, H2), _f32),
)


def _mid2_body(agg_ref, xs_ref, dinv_ref, b_ref, out_ref):
    dinv = jnp.transpose(dinv_ref[...])
    t = agg_ref[0] + agg_ref[1] + xs_ref[...]
    h = jnp.maximum(dinv * t + b_ref[...], 0.0)
    out_ref[...] = dinv * h


_mid2 = pl.pallas_call(
    _mid2_body,
    out_shape=jax.ShapeDtypeStruct((N_PAD, H2), _f32),
)


def _final_body(agg_ref, xs_ref, dinv_ref, w_ref, b_ref, out_ref):
    dinv = jnp.transpose(dinv_ref[...])
    t = agg_ref[0] + agg_ref[1] + xs_ref[...]
    z = jnp.dot(dinv * t, w_ref[...],
                preferred_element_type=_f32) + b_ref[...]
    m = jnp.max(z, axis=1, keepdims=True)
    e = jnp.exp(z - m)
    ls = (z - m) - jnp.log(jnp.sum(e, axis=1, keepdims=True))
    out_ref[...] = ls[:N]


_final = pl.pallas_call(
    _final_body,
    out_shape=jax.ShapeDtypeStruct((N, D_OUT), _f32),
)


def kernel(x, edge_index, W1, b1, W2, b2, W3, b3):
    src_p, dst_p = _edges(edge_index)
    src_p = src_p.reshape(NW * STEPS, W)
    dst_p = dst_p.reshape(NW * STEPS, W)
    x_p = jnp.pad(x, ((0, N_PAD - N), (0, 0)))

    degp = _hist(dst_p).reshape(NC, N_PAD)
    dinv, xs1 = _prep(degp, x_p, W1)
    agg1 = _agg32(xs1, src_p, dst_p)
    xs2 = _mid1(agg1, xs1, dinv, b1.reshape(1, H1), W2)
    agg2 = _agg16(xs2, src_p, dst_p)
    xs3 = _mid2(agg2, xs2, dinv, b2.reshape(1, H2))
    agg3 = _agg16(xs3, src_p, dst_p)
    return _final(agg3, xs3, dinv, W3, b3.reshape(1, D_OUT))


# speedup vs baseline: 1.0612x; 1.0612x over previous
import functools

import jax
import jax.numpy as jnp
from jax import lax
from jax.experimental import pallas as pl
from jax.experimental.pallas import tpu as pltpu
from jax.experimental.pallas import tpu_sc as plsc

N = 10000
E = 320000
D_IN = 128
H1 = 32
H2 = 16
D_OUT = 2

NC = 2
NS = 16
NW = NC * NS

W = 128
STEPS = 80
NBUF = 4
GROUPS = STEPS // NBUF
EPW = W * STEPS
E_PAD = EPW * NW
N_PAD = 10112
RPT = N_PAD // NS
N_SPARE = N_PAD - N

_f32 = jnp.float32

_SC_PARAMS = pltpu.CompilerParams(use_tc_tiling_on_sc=False)


def _mesh():
    return plsc.VectorSubcoreMesh(core_axis_name="c", subcore_axis_name="s")


def _hist_body(dst_hbm, out_hbm, acc_sp, didx, ones, zeros, vbuf, sem0, sem1,
               sem2, sem3):
    c = lax.axis_index("c")
    s = lax.axis_index("s")
    wid = s * NC + c
    rbase = s * RPT
    ssem = [sem0, sem1, sem2, sem3]
    pltpu.sync_copy(dst_hbm.at[pl.ds(wid * STEPS, STEPS)], didx)
    for j in range(W // 16):
        ones[pl.ds(j * 16, 16)] = jnp.ones((16,), _f32)
        zeros[pl.ds(j * 16, 16)] = jnp.zeros((16,), _f32)
    for k in range(RPT // W):
        pltpu.sync_copy(zeros, acc_sp.at[pl.ds(rbase + k * W, W)])
    pltpu.sync_copy(zeros.at[pl.ds(0, RPT % W)],
                    acc_sp.at[pl.ds(rbase + (RPT // W) * W, RPT % W)])
    plsc.subcore_barrier()

    def group(g, carry):
        for b in range(NBUF):
            t = g * NBUF + b

            @pl.when(g > 0)
            def _():
                pltpu.make_async_copy(ones, acc_sp.at[didx.at[t]],
                                      ssem[b]).wait()

            pltpu.async_copy(ones, acc_sp.at[didx.at[t]], ssem[b], add=True)
        return carry

    lax.fori_loop(0, GROUPS, group, 0)
    for b in range(NBUF):
        pltpu.make_async_copy(ones, acc_sp.at[didx.at[0]], ssem[b]).wait()
    plsc.subcore_barrier()
    obase = pl.multiple_of(c * N_PAD + rbase, 8)
    for k in range(RPT // W + 1):
        ln = W if k < RPT // W else RPT % W
        pltpu.sync_copy(acc_sp.at[pl.ds(rbase + k * W, ln)], vbuf.at[pl.ds(0, ln)])
        pltpu.sync_copy(vbuf.at[pl.ds(0, ln)], out_hbm.at[pl.ds(obase + k * W, ln)])


_hist = functools.partial(
    pl.kernel,
    out_type=jax.ShapeDtypeStruct((NC * N_PAD,), _f32),
    mesh=_mesh(),
    compiler_params=_SC_PARAMS,
    scratch_types=[
        pltpu.VMEM_SHARED((N_PAD,), _f32),
        pltpu.VMEM((STEPS, W), jnp.int32),
        pltpu.VMEM((W,), _f32),
        pltpu.VMEM((W,), _f32),
        pltpu.VMEM((W,), _f32),
        pltpu.SemaphoreType.DMA,
        pltpu.SemaphoreType.DMA,
        pltpu.SemaphoreType.DMA,
        pltpu.SemaphoreType.DMA,
    ],
)(_hist_body)


ANBUF = 8
LOOK = ANBUF // 2
AGROUPS = STEPS // ANBUF


def _agg_body_for(D):
    def agg(xs_hbm, src_hbm, dst_hbm, out_hbm, xs_sp, acc_sp, sidx, didx,
            *bufs):
        c = lax.axis_index("c")
        s = lax.axis_index("s")
        wid = s * NC + c
        rbase = s * RPT
        rows = list(bufs[:ANBUF])
        gsem = list(bufs[ANBUF:2 * ANBUF])
        ssem = list(bufs[2 * ANBUF:3 * ANBUF])
        pltpu.async_copy(xs_hbm.at[pl.ds(rbase, RPT)],
                         xs_sp.at[pl.ds(rbase, RPT)], gsem[0])
        pltpu.async_copy(src_hbm.at[pl.ds(wid * STEPS, STEPS)], sidx, gsem[1])
        pltpu.async_copy(dst_hbm.at[pl.ds(wid * STEPS, STEPS)], didx, gsem[2])

        def zrow(i, carry):
            for j in range(D // 16):
                rows[0][i, pl.ds(j * 16, 16)] = jnp.zeros((16,), _f32)
            return carry

        lax.fori_loop(0, W, zrow, 0)
        pltpu.make_async_copy(xs_hbm.at[pl.ds(rbase, RPT)],
                              xs_sp.at[pl.ds(rbase, RPT)], gsem[0]).wait()
        pltpu.make_async_copy(src_hbm.at[pl.ds(wid * STEPS, STEPS)], sidx,
                              gsem[1]).wait()
        pltpu.make_async_copy(dst_hbm.at[pl.ds(wid * STEPS, STEPS)], didx,
                              gsem[2]).wait()
        for k in range(RPT // W):
            pltpu.sync_copy(rows[0], acc_sp.at[pl.ds(rbase + k * W, W)])
        pltpu.sync_copy(rows[0].at[pl.ds(0, RPT % W)],
                        acc_sp.at[pl.ds(rbase + (RPT // W) * W, RPT % W)])
        plsc.subcore_barrier()

        for t0 in range(LOOK):
            pltpu.async_copy(xs_sp.at[sidx.at[t0]], rows[t0], gsem[t0])

        def group(g, carry):
            for b in range(ANBUF):
                t = g * ANBUF + b
                bp = (b + LOOK) % ANBUF
                pltpu.make_async_copy(xs_sp.at[sidx.at[t]], rows[b],
                                      gsem[b]).wait()
                pltpu.async_copy(rows[b], acc_sp.at[didx.at[t]], ssem[b],
                                 add=True)
                if b < ANBUF - LOOK:
                    @pl.when(g > 0)
                    def _():
                        pltpu.make_async_copy(rows[bp], acc_sp.at[didx.at[t]],
                                              ssem[bp]).wait()

                    pltpu.async_copy(xs_sp.at[sidx.at[t + LOOK]], rows[bp],
                                     gsem[bp])
                else:
                    @pl.when(g < AGROUPS - 1)
                    def _():
                        pltpu.make_async_copy(rows[bp], acc_sp.at[didx.at[t]],
                                              ssem[bp]).wait()
                        pltpu.async_copy(xs_sp.at[sidx.at[t + LOOK]], rows[bp],
                                         gsem[bp])
            return carry

        lax.fori_loop(0, AGROUPS, group, 0)
        for b in range(ANBUF):
            pltpu.make_async_copy(rows[b], acc_sp.at[didx.at[0]],
                                  ssem[b]).wait()
        plsc.subcore_barrier()
        pltpu.sync_copy(acc_sp.at[pl.ds(rbase, RPT)],
                        out_hbm.at[c, pl.ds(rbase, RPT)])

    return agg


def _make_agg(D):
    return functools.partial(
        pl.kernel,
        out_type=jax.ShapeDtypeStruct((NC, N_PAD, D), _f32),
        mesh=_mesh(),
        compiler_params=_SC_PARAMS,
        scratch_types=[
            pltpu.VMEM_SHARED((N_PAD, D), _f32),
            pltpu.VMEM_SHARED((N_PAD, D), _f32),
            pltpu.VMEM((STEPS, W), jnp.int32),
            pltpu.VMEM((STEPS, W), jnp.int32),
        ] + [pltpu.VMEM((W, D), _f32)] * ANBUF
          + [pltpu.SemaphoreType.DMA] * (2 * ANBUF),
    )(_agg_body_for(D))


_agg32 = _make_agg(H1)
_agg16 = _make_agg(H2)


def _edges_body(ei_ref, src_ref, dst_ref):
    pad = N + lax.rem(
        lax.broadcasted_iota(jnp.int32, (E_PAD - E,), 0), N_SPARE)
    src_ref[pl.ds(0, E)] = ei_ref[0]
    dst_ref[pl.ds(0, E)] = ei_ref[1]
    src_ref[pl.ds(E, E_PAD - E)] = pad
    dst_ref[pl.ds(E, E_PAD - E)] = pad


_edges = pl.pallas_call(
    _edges_body,
    out_shape=(jax.ShapeDtypeStruct((E_PAD,), jnp.int32),
               jax.ShapeDtypeStruct((E_PAD,), jnp.int32)),
)


def _prep_body(degp_ref, x_ref, w1_ref, dinv_ref, xs1_ref):
    deg = jnp.sum(degp_ref[...], axis=0, keepdims=True) + 1.0
    dinv = lax.rsqrt(deg)
    mask = (lax.broadcasted_iota(jnp.int32, (1, N_PAD), 1) < N).astype(_f32)
    dinv = dinv * mask
    dinv_ref[...] = dinv
    dinv_col = jnp.transpose(dinv)
    xs1_ref[...] = jnp.dot(x_ref[...], w1_ref[...],
                           preferred_element_type=_f32) * dinv_col


_prep = pl.pallas_call(
    _prep_body,
    out_shape=(jax.ShapeDtypeStruct((1, N_PAD), _f32),
               jax.ShapeDtypeStruct((N_PAD, H1), _f32)),
)


def _mid1_body(agg_ref, xs_ref, dinv_ref, b_ref, w_ref, out_ref):
    dinv = jnp.transpose(dinv_ref[...])
    t = agg_ref[0] + agg_ref[1] + xs_ref[...]
    h = jnp.maximum(dinv * t + b_ref[...], 0.0)
    out_ref[...] = jnp.dot(h, w_ref[...], preferred_element_type=_f32) * dinv


_mid1 = pl.pallas_call(
    _mid1_body,
    out_shape=jax.ShapeDtypeStruct((N_PAD, H2), _f32),
)


def _mid2_body(agg_ref, xs_ref, dinv_ref, b_ref, out_ref):
    dinv = jnp.transpose(dinv_ref[...])
    t = agg_ref[0] + agg_ref[1] + xs_ref[...]
    h = jnp.maximum(dinv * t + b_ref[...], 0.0)
    out_ref[...] = dinv * h


_mid2 = pl.pallas_call(
    _mid2_body,
    out_shape=jax.ShapeDtypeStruct((N_PAD, H2), _f32),
)


def _final_body(agg_ref, xs_ref, dinv_ref, w_ref, b_ref, out_ref):
    dinv = jnp.transpose(dinv_ref[...])
    t = agg_ref[0] + agg_ref[1] + xs_ref[...]
    z = jnp.dot(dinv * t, w_ref[...],
                preferred_element_type=_f32) + b_ref[...]
    m = jnp.max(z, axis=1, keepdims=True)
    e = jnp.exp(z - m)
    ls = (z - m) - jnp.log(jnp.sum(e, axis=1, keepdims=True))
    out_ref[...] = ls[:N]


_final = pl.pallas_call(
    _final_body,
    out_shape=jax.ShapeDtypeStruct((N, D_OUT), _f32),
)


def kernel(x, edge_index, W1, b1, W2, b2, W3, b3):
    src_p, dst_p = _edges(edge_index)
    src_p = src_p.reshape(NW * STEPS, W)
    dst_p = dst_p.reshape(NW * STEPS, W)
    x_p = jnp.pad(x, ((0, N_PAD - N), (0, 0)))

    degp = _hist(dst_p).reshape(NC, N_PAD)
    dinv, xs1 = _prep(degp, x_p, W1)
    agg1 = _agg32(xs1, src_p, dst_p)
    xs2 = _mid1(agg1, xs1, dinv, b1.reshape(1, H1), W2)
    agg2 = _agg16(xs2, src_p, dst_p)
    xs3 = _mid2(agg2, xs2, dinv, b2.reshape(1, H2))
    agg3 = _agg16(xs3, src_p, dst_p)
    return _final(agg3, xs3, dinv, W3, b3.reshape(1, D_OUT))


# speedup vs baseline: 1.4107x; 1.3294x over previous
import functools

import jax
import jax.numpy as jnp
from jax import lax
from jax.experimental import pallas as pl
from jax.experimental.pallas import tpu as pltpu
from jax.experimental.pallas import tpu_sc as plsc

N = 10000
E = 320000
D_IN = 128
H1 = 32
H2 = 16
D_OUT = 2

NC = 2
NS = 16
NW = NC * NS

W = 128
STEPS = 80
HSTEPS = STEPS * NC
EPW = W * STEPS
E_PAD = EPW * NW
N_PAD = 10112
RPT = N_PAD // NS
N_SPARE = N_PAD - N
P4R = N_PAD // 4
P8R = N_PAD // 8

_f32 = jnp.float32

_SC_PARAMS = pltpu.CompilerParams(use_tc_tiling_on_sc=False,
                                  needs_layout_passes=False)


def _mesh():
    return plsc.VectorSubcoreMesh(core_axis_name="c", subcore_axis_name="s")


def _hist_body(dst_hbm, dinv_hbm, b32_hbm, b16_hbm, acc_sp, didx, ones, zeros,
               vbuf, dinvv, b32, b16, sem0, sem1, sem2, sem3):
    c = lax.axis_index("c")
    s = lax.axis_index("s")
    rbase = s * RPT
    ssem = [sem0, sem1, sem2, sem3]
    pltpu.sync_copy(dst_hbm.at[pl.ds(s * HSTEPS, HSTEPS)], didx)
    for j in range(W // 16):
        ones[pl.ds(j * 16, 16)] = jnp.ones((16,), _f32)
        zeros[pl.ds(j * 16, 16)] = jnp.zeros((16,), _f32)
    for k in range(RPT // W):
        pltpu.sync_copy(zeros, acc_sp.at[pl.ds(rbase + k * W, W)])
    pltpu.sync_copy(zeros.at[pl.ds(0, RPT % W)],
                    acc_sp.at[pl.ds(rbase + (RPT // W) * W, RPT % W)])
    plsc.subcore_barrier()

    def group(g, carry):
        for b in range(4):
            t = g * 4 + b

            @pl.when(g > 0)
            def _():
                pltpu.make_async_copy(ones, acc_sp.at[didx.at[t]],
                                      ssem[b]).wait()

            pltpu.async_copy(ones, acc_sp.at[didx.at[t]], ssem[b], add=True)
        return carry

    lax.fori_loop(0, HSTEPS // 4, group, 0)
    for b in range(4):
        pltpu.make_async_copy(ones, acc_sp.at[didx.at[0]], ssem[b]).wait()
    plsc.subcore_barrier()

    for k in range(RPT // W + 1):
        ln = W if k < RPT // W else RPT % W
        pltpu.sync_copy(acc_sp.at[pl.ds(rbase + k * W, ln)],
                        dinvv.at[pl.ds(k * W, ln)])

    def newton(k, carry):
        deg = dinvv[pl.ds(k * 16, 16)] + 1.0
        i = plsc.bitcast(deg, jnp.int32)
        y = plsc.bitcast(0x5F3759DF - lax.shift_right_logical(i, 1), _f32)
        half = deg * 0.5
        y = y * (1.5 - half * y * y)
        y = y * (1.5 - half * y * y)
        y = y * (1.5 - half * y * y)
        ridx = rbase + k * 16 + lax.iota(jnp.int32, 16)
        y = jnp.where(ridx < N, y, 0.0)
        dinvv[pl.ds(k * 16, 16)] = y
        return carry

    lax.fori_loop(0, RPT // 16 + 1, newton, 0)

    def bcast(r, carry):
        v = plsc.load_gather(dinvv, [jnp.full((16,), r, jnp.int32)])
        b32[r, pl.ds(0, 16)] = v
        b32[r, pl.ds(16, 16)] = v
        b16[r, pl.ds(0, 16)] = v
        return carry

    lax.fori_loop(0, RPT, bcast, 0)

    @pl.when(c == 0)
    def _():
        pltpu.sync_copy(dinvv.at[pl.ds(0, RPT)],
                        dinv_hbm.at[pl.ds(rbase, RPT)])
        pltpu.sync_copy(b32, b32_hbm.at[pl.ds(rbase, RPT)])
        pltpu.sync_copy(b16, b16_hbm.at[pl.ds(rbase, RPT)])


_hist = functools.partial(
    pl.kernel,
    out_type=(jax.ShapeDtypeStruct((N_PAD,), _f32),
              jax.ShapeDtypeStruct((N_PAD, H1), _f32),
              jax.ShapeDtypeStruct((N_PAD, H2), _f32)),
    mesh=_mesh(),
    compiler_params=_SC_PARAMS,
    scratch_types=[
        pltpu.VMEM_SHARED((N_PAD,), _f32),
        pltpu.VMEM((HSTEPS, W), jnp.int32),
        pltpu.VMEM((W,), _f32),
        pltpu.VMEM((W,), _f32),
        pltpu.VMEM((W,), _f32),
        pltpu.VMEM((RPT + 16,), _f32),
        pltpu.VMEM((RPT, H1), _f32),
        pltpu.VMEM((RPT, H2), _f32),
        pltpu.SemaphoreType.DMA,
        pltpu.SemaphoreType.DMA,
        pltpu.SemaphoreType.DMA,
        pltpu.SemaphoreType.DMA,
    ],
)(_hist_body)


ANBUF = 8
LOOK = ANBUF // 2
AGROUPS = STEPS // ANBUF


def _agg_body_for(D, scale):
    def agg(*refs):
        if scale:
            (xs_hbm, dinv_hbm, src_hbm, dst_hbm, out_hbm, xs_sp, acc_sp,
             sidx, didx, xbuf, dinvv, *bufs) = refs
        else:
            (xs_hbm, src_hbm, dst_hbm, out_hbm, xs_sp, acc_sp,
             sidx, didx, *bufs) = refs
        c = lax.axis_index("c")
        s = lax.axis_index("s")
        wid = s * NC + c
        rbase = s * RPT
        rows = list(bufs[:ANBUF])
        gsem = list(bufs[ANBUF:2 * ANBUF])
        ssem = list(bufs[2 * ANBUF:3 * ANBUF])
        if scale:
            pltpu.async_copy(xs_hbm.at[pl.ds(rbase, RPT)], xbuf, gsem[0])
            pltpu.async_copy(dinv_hbm.at[pl.ds(rbase, RPT)], dinvv, gsem[3])
        else:
            pltpu.async_copy(xs_hbm.at[pl.ds(rbase, RPT)],
                             xs_sp.at[pl.ds(rbase, RPT)], gsem[0])
        pltpu.async_copy(src_hbm.at[pl.ds(wid * STEPS, STEPS)], sidx, gsem[1])
        pltpu.async_copy(dst_hbm.at[pl.ds(wid * STEPS, STEPS)], didx, gsem[2])

        def zrow(i, carry):
            for j in range(D // 16):
                rows[0][i, pl.ds(j * 16, 16)] = jnp.zeros((16,), _f32)
            return carry

        lax.fori_loop(0, W, zrow, 0)
        if scale:
            pltpu.make_async_copy(xs_hbm.at[pl.ds(rbase, RPT)], xbuf,
                                  gsem[0]).wait()
            pltpu.make_async_copy(dinv_hbm.at[pl.ds(rbase, RPT)], dinvv,
                                  gsem[3]).wait()

            def srow(r, carry):
                dv = plsc.load_gather(dinvv,
                                      [jnp.full((16,), r, jnp.int32)])
                for j in range(D // 16):
                    xbuf[r, pl.ds(j * 16, 16)] = xbuf[r, pl.ds(j * 16, 16)] * dv
                return carry

            lax.fori_loop(0, RPT, srow, 0)
            pltpu.sync_copy(xbuf, xs_sp.at[pl.ds(rbase, RPT)])
        else:
            pltpu.make_async_copy(xs_hbm.at[pl.ds(rbase, RPT)],
                                  xs_sp.at[pl.ds(rbase, RPT)], gsem[0]).wait()
        pltpu.make_async_copy(src_hbm.at[pl.ds(wid * STEPS, STEPS)], sidx,
                              gsem[1]).wait()
        pltpu.make_async_copy(dst_hbm.at[pl.ds(wid * STEPS, STEPS)], didx,
                              gsem[2]).wait()
        for k in range(RPT // W):
            pltpu.sync_copy(rows[0], acc_sp.at[pl.ds(rbase + k * W, W)])
        pltpu.sync_copy(rows[0].at[pl.ds(0, RPT % W)],
                        acc_sp.at[pl.ds(rbase + (RPT // W) * W, RPT % W)])
        plsc.subcore_barrier()

        for t0 in range(LOOK):
            pltpu.async_copy(xs_sp.at[sidx.at[t0]], rows[t0], gsem[t0])

        def group(g, carry):
            for b in range(ANBUF):
                t = g * ANBUF + b
                bp = (b + LOOK) % ANBUF
                pltpu.make_async_copy(xs_sp.at[sidx.at[t]], rows[b],
                                      gsem[b]).wait()
                pltpu.async_copy(rows[b], acc_sp.at[didx.at[t]], ssem[b],
                                 add=True)
                if b < ANBUF - LOOK:
                    @pl.when(g > 0)
                    def _():
                        pltpu.make_async_copy(rows[bp], acc_sp.at[didx.at[t]],
                                              ssem[bp]).wait()

                    pltpu.async_copy(xs_sp.at[sidx.at[t + LOOK]], rows[bp],
                                     gsem[bp])
                else:
                    @pl.when(g < AGROUPS - 1)
                    def _():
                        pltpu.make_async_copy(rows[bp], acc_sp.at[didx.at[t]],
                                              ssem[bp]).wait()
                        pltpu.async_copy(xs_sp.at[sidx.at[t + LOOK]], rows[bp],
                                         gsem[bp])
            return carry

        lax.fori_loop(0, AGROUPS, group, 0)
        for b in range(ANBUF):
            pltpu.make_async_copy(rows[b], acc_sp.at[didx.at[0]],
                                  ssem[b]).wait()
        plsc.subcore_barrier()
        pltpu.sync_copy(acc_sp.at[pl.ds(rbase, RPT)],
                        out_hbm.at[c, pl.ds(rbase, RPT)])

    return agg


def _make_agg(D, scale):
    extra = ([pltpu.VMEM((RPT, D), _f32),
              pltpu.VMEM((RPT,), _f32)]
             if scale else [])
    return functools.partial(
        pl.kernel,
        out_type=jax.ShapeDtypeStruct((NC, N_PAD, D), _f32),
        mesh=_mesh(),
        compiler_params=_SC_PARAMS,
        scratch_types=[
            pltpu.VMEM_SHARED((N_PAD, D), _f32),
            pltpu.VMEM_SHARED((N_PAD, D), _f32),
            pltpu.VMEM((STEPS, W), jnp.int32),
            pltpu.VMEM((STEPS, W), jnp.int32),
        ] + extra
          + [pltpu.VMEM((W, D), _f32)] * ANBUF
          + [pltpu.SemaphoreType.DMA] * (2 * ANBUF),
    )(_agg_body_for(D, scale))


_agg32s = _make_agg(H1, True)
_agg16s = _make_agg(H2, True)
_agg16n = _make_agg(H2, False)


def _edges_body(ei_ref, src_ref, dst_ref):
    pad = N + lax.rem(
        lax.broadcasted_iota(jnp.int32, (E_PAD - E,), 0), N_SPARE)
    src_ref[pl.ds(0, E)] = ei_ref[0]
    dst_ref[pl.ds(0, E)] = ei_ref[1]
    src_ref[pl.ds(E, E_PAD - E)] = pad
    dst_ref[pl.ds(E, E_PAD - E)] = pad


_edges = pl.pallas_call(
    _edges_body,
    out_shape=(jax.ShapeDtypeStruct((E_PAD,), jnp.int32),
               jax.ShapeDtypeStruct((E_PAD,), jnp.int32)),
)


def _prep_body(x_ref, w1_ref, out_ref):
    out_ref[...] = jnp.dot(x_ref[...], w1_ref[...], preferred_element_type=_f32)


_prep = pl.pallas_call(
    _prep_body,
    out_shape=jax.ShapeDtypeStruct((P4R, 128), _f32),
)


def _mid1_body(agg_ref, xw_ref, dv_ref, b_ref, w_ref, out_ref):
    dv = dv_ref[...]
    t = agg_ref[0] + agg_ref[1] + dv * xw_ref[...]
    h = jnp.maximum(dv * t + b_ref[...], 0.0)
    out_ref[...] = jnp.dot(h, w_ref[...], preferred_element_type=_f32)


_mid1 = pl.pallas_call(
    _mid1_body,
    out_shape=jax.ShapeDtypeStruct((P4R, 64), _f32),
)


def _mid2_body(agg_ref, xw_ref, dv_ref, b_ref, out_ref):
    dv = dv_ref[...]
    t = agg_ref[0] + agg_ref[1] + dv * xw_ref[...]
    out_ref[...] = dv * jnp.maximum(dv * t + b_ref[...], 0.0)


_mid2 = pl.pallas_call(
    _mid2_body,
    out_shape=jax.ShapeDtypeStruct((P8R, 128), _f32),
)


def _final_body(agg_ref, xs_ref, dv_ref, w_ref, b_ref, p_ref, out_ref):
    t = agg_ref[0] + agg_ref[1] + xs_ref[...]
    z = jnp.dot(dv_ref[...] * t, w_ref[...],
                preferred_element_type=_f32) + b_ref[...]
    zsw = jnp.dot(z, p_ref[...], preferred_element_type=_f32)
    m = jnp.maximum(z, zsw)
    e = jnp.exp(z - m)
    es = e + jnp.dot(e, p_ref[...], preferred_element_type=_f32)
    out_ref[...] = (z - m) - jnp.log(es)


_final = pl.pallas_call(
    _final_body,
    out_shape=jax.ShapeDtypeStruct((P8R, 16), _f32),
)


def kernel(x, edge_index, W1, b1, W2, b2, W3, b3):
    src_p, dst_p = _edges(edge_index)
    src_p = src_p.reshape(NW * STEPS, W)
    dst_p = dst_p.reshape(NW * STEPS, W)
    x_p4 = jnp.pad(x, ((0, N_SPARE), (0, 0))).reshape(P4R, 4 * D_IN)
    w1b = jax.scipy.linalg.block_diag(W1, W1, W1, W1)
    w2b = jax.scipy.linalg.block_diag(W2, W2, W2, W2)
    w3b = jax.scipy.linalg.block_diag(*([W3] * 8))
    b1t = jnp.tile(b1, 4).reshape(1, 128)
    b2t = jnp.tile(b2, 8).reshape(1, 128)
    b3t = jnp.tile(b3, 8).reshape(1, 16)
    pswap = jnp.kron(jnp.eye(8, dtype=_f32),
                     jnp.array([[0.0, 1.0], [1.0, 0.0]], _f32))

    dinv1, db32, db16 = _hist(dst_p)
    xw1 = _prep(x_p4, w1b)
    agg1 = _agg32s(xw1.reshape(N_PAD, H1), dinv1, src_p, dst_p)
    xw2 = _mid1(agg1.reshape(NC, P4R, 128), xw1,
                db32.reshape(P4R, 128), b1t, w2b)
    xw2l = xw2.reshape(N_PAD, H2)
    agg2 = _agg16s(xw2l, dinv1, src_p, dst_p)
    xs3 = _mid2(agg2.reshape(NC, P8R, 128), xw2l.reshape(P8R, 128),
                db16.reshape(P8R, 128), b2t)
    agg3 = _agg16n(xs3.reshape(N_PAD, H2), src_p, dst_p)
    ls = _final(agg3.reshape(NC, P8R, 128), xs3, db16.reshape(P8R, 128),
                w3b, b3t, pswap)
    return ls.reshape(N_PAD, D_OUT)[:N]


# speedup vs baseline: 1.4278x; 1.0121x over previous
import functools

import jax
import jax.numpy as jnp
from jax import lax
from jax.experimental import pallas as pl
from jax.experimental.pallas import tpu as pltpu
from jax.experimental.pallas import tpu_sc as plsc

N = 10000
E = 320000
D_IN = 128
H1 = 32
H2 = 16
D_OUT = 2

NC = 2
NS = 16
NW = NC * NS

W = 128
STEPS = 80
HSTEPS = STEPS * NC
EPW = W * STEPS
E_PAD = EPW * NW
N_PAD = 10112
RPT = N_PAD // NS
N_SPARE = N_PAD - N
P4R = N_PAD // 4
P8R = N_PAD // 8

_f32 = jnp.float32

_SC_PARAMS = pltpu.CompilerParams(use_tc_tiling_on_sc=False,
                                  needs_layout_passes=False)


def _mesh():
    return plsc.VectorSubcoreMesh(core_axis_name="c", subcore_axis_name="s")


def _hist_body(dst_hbm, dinv_hbm, b32_hbm, b16_hbm, acc_sp, didx, ones, zeros,
               vbuf, dinvv, b32, b16, sem0, sem1, sem2, sem3):
    c = lax.axis_index("c")
    s = lax.axis_index("s")
    rbase = s * RPT
    ssem = [sem0, sem1, sem2, sem3]
    pltpu.sync_copy(dst_hbm.at[pl.ds(s * HSTEPS, HSTEPS)], didx)
    for j in range(W // 16):
        ones[pl.ds(j * 16, 16)] = jnp.ones((16,), _f32)
        zeros[pl.ds(j * 16, 16)] = jnp.zeros((16,), _f32)
    for k in range(RPT // W):
        pltpu.sync_copy(zeros, acc_sp.at[pl.ds(rbase + k * W, W)])
    pltpu.sync_copy(zeros.at[pl.ds(0, RPT % W)],
                    acc_sp.at[pl.ds(rbase + (RPT // W) * W, RPT % W)])
    plsc.subcore_barrier()

    def group(g, carry):
        for b in range(4):
            t = g * 4 + b

            @pl.when(g > 0)
            def _():
                pltpu.make_async_copy(ones, acc_sp.at[didx.at[t]],
                                      ssem[b]).wait()

            pltpu.async_copy(ones, acc_sp.at[didx.at[t]], ssem[b], add=True)
        return carry

    lax.fori_loop(0, HSTEPS // 4, group, 0)
    for b in range(4):
        pltpu.make_async_copy(ones, acc_sp.at[didx.at[0]], ssem[b]).wait()
    plsc.subcore_barrier()

    for k in range(RPT // W + 1):
        ln = W if k < RPT // W else RPT % W
        pltpu.sync_copy(acc_sp.at[pl.ds(rbase + k * W, ln)],
                        dinvv.at[pl.ds(k * W, ln)])

    def newton(k, carry):
        deg = dinvv[pl.ds(k * 16, 16)] + 1.0
        i = plsc.bitcast(deg, jnp.int32)
        y = plsc.bitcast(0x5F3759DF - lax.shift_right_logical(i, 1), _f32)
        half = deg * 0.5
        y = y * (1.5 - half * y * y)
        y = y * (1.5 - half * y * y)
        y = y * (1.5 - half * y * y)
        ridx = rbase + k * 16 + lax.iota(jnp.int32, 16)
        y = jnp.where(ridx < N, y, 0.0)
        dinvv[pl.ds(k * 16, 16)] = y
        return carry

    lax.fori_loop(0, RPT // 16 + 1, newton, 0)

    def bcast(r, carry):
        v = plsc.load_gather(dinvv, [jnp.full((16,), r, jnp.int32)])
        b32[r, pl.ds(0, 16)] = v
        b32[r, pl.ds(16, 16)] = v
        b16[r, pl.ds(0, 16)] = v
        return carry

    lax.fori_loop(0, RPT, bcast, 0)

    @pl.when(c == 0)
    def _():
        pltpu.sync_copy(dinvv.at[pl.ds(0, RPT)],
                        dinv_hbm.at[pl.ds(rbase, RPT)])
        pltpu.sync_copy(b32, b32_hbm.at[pl.ds(rbase, RPT)])
        pltpu.sync_copy(b16, b16_hbm.at[pl.ds(rbase, RPT)])


_hist = functools.partial(
    pl.kernel,
    out_type=(jax.ShapeDtypeStruct((N_PAD,), _f32),
              jax.ShapeDtypeStruct((N_PAD, H1), _f32),
              jax.ShapeDtypeStruct((N_PAD, H2), _f32)),
    mesh=_mesh(),
    compiler_params=_SC_PARAMS,
    scratch_types=[
        pltpu.VMEM_SHARED((N_PAD,), _f32),
        pltpu.VMEM((HSTEPS, W), jnp.int32),
        pltpu.VMEM((W,), _f32),
        pltpu.VMEM((W,), _f32),
        pltpu.VMEM((W,), _f32),
        pltpu.VMEM((RPT + 16,), _f32),
        pltpu.VMEM((RPT, H1), _f32),
        pltpu.VMEM((RPT, H2), _f32),
        pltpu.SemaphoreType.DMA,
        pltpu.SemaphoreType.DMA,
        pltpu.SemaphoreType.DMA,
        pltpu.SemaphoreType.DMA,
    ],
)(_hist_body)


ANBUF = 8
LOOK = ANBUF // 2
AGROUPS = STEPS // ANBUF


def _agg_body_for(D, scale):
    def agg(*refs):
        if scale:
            (xs_hbm, dinv_hbm, src_hbm, dst_hbm, out_hbm, xs_sp, acc_sp,
             sidx, didx, xbuf, dinvv, *bufs) = refs
        else:
            (xs_hbm, src_hbm, dst_hbm, out_hbm, xs_sp, acc_sp,
             sidx, didx, *bufs) = refs
        c = lax.axis_index("c")
        s = lax.axis_index("s")
        wid = s * NC + c
        rbase = s * RPT
        rows = list(bufs[:ANBUF])
        gsem = list(bufs[ANBUF:2 * ANBUF])
        ssem = list(bufs[2 * ANBUF:3 * ANBUF])
        if scale:
            pltpu.async_copy(xs_hbm.at[pl.ds(rbase, RPT)], xbuf, gsem[0])
            pltpu.async_copy(dinv_hbm.at[pl.ds(rbase, RPT)], dinvv, gsem[3])
        else:
            pltpu.async_copy(xs_hbm.at[pl.ds(rbase, RPT)],
                             xs_sp.at[pl.ds(rbase, RPT)], gsem[0])
        pltpu.async_copy(src_hbm.at[pl.ds(wid * STEPS, STEPS)], sidx, gsem[1])
        pltpu.async_copy(dst_hbm.at[pl.ds(wid * STEPS, STEPS)], didx, gsem[2])

        def zrow(i, carry):
            for j in range(D // 16):
                rows[0][i, pl.ds(j * 16, 16)] = jnp.zeros((16,), _f32)
            return carry

        lax.fori_loop(0, W, zrow, 0)
        for k in range(RPT // W):
            pltpu.async_copy(rows[0], acc_sp.at[pl.ds(rbase + k * W, W)],
                             ssem[k])
        pltpu.async_copy(rows[0].at[pl.ds(0, RPT % W)],
                         acc_sp.at[pl.ds(rbase + (RPT // W) * W, RPT % W)],
                         ssem[RPT // W])
        if scale:
            pltpu.make_async_copy(xs_hbm.at[pl.ds(rbase, RPT)], xbuf,
                                  gsem[0]).wait()
            pltpu.make_async_copy(dinv_hbm.at[pl.ds(rbase, RPT)], dinvv,
                                  gsem[3]).wait()

            def srow(r, carry):
                dv = plsc.load_gather(dinvv,
                                      [jnp.full((16,), r, jnp.int32)])
                for j in range(D // 16):
                    xbuf[r, pl.ds(j * 16, 16)] = xbuf[r, pl.ds(j * 16, 16)] * dv
                return carry

            lax.fori_loop(0, RPT, srow, 0)
            pltpu.sync_copy(xbuf, xs_sp.at[pl.ds(rbase, RPT)])
        else:
            pltpu.make_async_copy(xs_hbm.at[pl.ds(rbase, RPT)],
                                  xs_sp.at[pl.ds(rbase, RPT)], gsem[0]).wait()
        pltpu.make_async_copy(src_hbm.at[pl.ds(wid * STEPS, STEPS)], sidx,
                              gsem[1]).wait()
        pltpu.make_async_copy(dst_hbm.at[pl.ds(wid * STEPS, STEPS)], didx,
                              gsem[2]).wait()
        for k in range(RPT // W):
            pltpu.make_async_copy(rows[0], acc_sp.at[pl.ds(rbase + k * W, W)],
                                  ssem[k]).wait()
        pltpu.make_async_copy(rows[0].at[pl.ds(0, RPT % W)],
                              acc_sp.at[pl.ds(rbase + (RPT // W) * W, RPT % W)],
                              ssem[RPT // W]).wait()
        plsc.subcore_barrier()

        for t0 in range(LOOK):
            pltpu.async_copy(xs_sp.at[sidx.at[t0]], rows[t0], gsem[t0])

        def group(g, carry):
            for b in range(ANBUF):
                t = g * ANBUF + b
                bp = (b + LOOK) % ANBUF
                pltpu.make_async_copy(xs_sp.at[sidx.at[t]], rows[b],
                                      gsem[b]).wait()
                pltpu.async_copy(rows[b], acc_sp.at[didx.at[t]], ssem[b],
                                 add=True)
                if b < ANBUF - LOOK:
                    @pl.when(g > 0)
                    def _():
                        pltpu.make_async_copy(rows[bp], acc_sp.at[didx.at[t]],
                                              ssem[bp]).wait()

                    pltpu.async_copy(xs_sp.at[sidx.at[t + LOOK]], rows[bp],
                                     gsem[bp])
                else:
                    @pl.when(g < AGROUPS - 1)
                    def _():
                        pltpu.make_async_copy(rows[bp], acc_sp.at[didx.at[t]],
                                              ssem[bp]).wait()
                        pltpu.async_copy(xs_sp.at[sidx.at[t + LOOK]], rows[bp],
                                         gsem[bp])
            return carry

        lax.fori_loop(0, AGROUPS, group, 0)
        for b in range(ANBUF):
            pltpu.make_async_copy(rows[b], acc_sp.at[didx.at[0]],
                                  ssem[b]).wait()
        plsc.subcore_barrier()
        pltpu.sync_copy(acc_sp.at[pl.ds(rbase, RPT)],
                        out_hbm.at[c, pl.ds(rbase, RPT)])

    return agg


def _make_agg(D, scale):
    extra = ([pltpu.VMEM((RPT, D), _f32),
              pltpu.VMEM((RPT,), _f32)]
             if scale else [])
    return functools.partial(
        pl.kernel,
        out_type=jax.ShapeDtypeStruct((NC, N_PAD, D), _f32),
        mesh=_mesh(),
        compiler_params=_SC_PARAMS,
        scratch_types=[
            pltpu.VMEM_SHARED((N_PAD, D), _f32),
            pltpu.VMEM_SHARED((N_PAD, D), _f32),
            pltpu.VMEM((STEPS, W), jnp.int32),
            pltpu.VMEM((STEPS, W), jnp.int32),
        ] + extra
          + [pltpu.VMEM((W, D), _f32)] * ANBUF
          + [pltpu.SemaphoreType.DMA] * (2 * ANBUF),
    )(_agg_body_for(D, scale))


_agg32s = _make_agg(H1, True)
_agg16s = _make_agg(H2, True)
_agg16n = _make_agg(H2, False)


def _edges_body(ei_ref, src_ref, dst_ref):
    pad = N + lax.rem(
        lax.broadcasted_iota(jnp.int32, (E_PAD - E,), 0), N_SPARE)
    src_ref[pl.ds(0, E)] = ei_ref[0]
    dst_ref[pl.ds(0, E)] = ei_ref[1]
    src_ref[pl.ds(E, E_PAD - E)] = pad
    dst_ref[pl.ds(E, E_PAD - E)] = pad


_edges = pl.pallas_call(
    _edges_body,
    out_shape=(jax.ShapeDtypeStruct((E_PAD,), jnp.int32),
               jax.ShapeDtypeStruct((E_PAD,), jnp.int32)),
)


def _prep_body(x_ref, w1_ref, out_ref):
    out_ref[...] = jnp.dot(x_ref[...], w1_ref[...], preferred_element_type=_f32)


_prep = pl.pallas_call(
    _prep_body,
    out_shape=jax.ShapeDtypeStruct((P4R, 128), _f32),
)


def _mid1_body(agg_ref, xw_ref, dv_ref, b_ref, w_ref, out_ref):
    dv = dv_ref[...]
    t = agg_ref[0] + agg_ref[1] + dv * xw_ref[...]
    h = jnp.maximum(dv * t + b_ref[...], 0.0)
    out_ref[...] = jnp.dot(h, w_ref[...], preferred_element_type=_f32)


_mid1 = pl.pallas_call(
    _mid1_body,
    out_shape=jax.ShapeDtypeStruct((P4R, 64), _f32),
)


def _mid2_body(agg_ref, xw_ref, dv_ref, b_ref, out_ref):
    dv = dv_ref[...]
    t = agg_ref[0] + agg_ref[1] + dv * xw_ref[...]
    out_ref[...] = dv * jnp.maximum(dv * t + b_ref[...], 0.0)


_mid2 = pl.pallas_call(
    _mid2_body,
    out_shape=jax.ShapeDtypeStruct((P8R, 128), _f32),
)


def _final_body(agg_ref, xs_ref, dv_ref, w_ref, b_ref, p_ref, out_ref):
    t = agg_ref[0] + agg_ref[1] + xs_ref[...]
    z = jnp.dot(dv_ref[...] * t, w_ref[...],
                preferred_element_type=_f32) + b_ref[...]
    zsw = jnp.dot(z, p_ref[...], preferred_element_type=_f32)
    m = jnp.maximum(z, zsw)
    e = jnp.exp(z - m)
    es = e + jnp.dot(e, p_ref[...], preferred_element_type=_f32)
    out_ref[...] = (z - m) - jnp.log(es)


_final = pl.pallas_call(
    _final_body,
    out_shape=jax.ShapeDtypeStruct((P8R, 16), _f32),
)


def kernel(x, edge_index, W1, b1, W2, b2, W3, b3):
    src_p, dst_p = _edges(edge_index)
    src_p = src_p.reshape(NW * STEPS, W)
    dst_p = dst_p.reshape(NW * STEPS, W)
    x_p4 = jnp.pad(x, ((0, N_SPARE), (0, 0))).reshape(P4R, 4 * D_IN)
    w1b = jax.scipy.linalg.block_diag(W1, W1, W1, W1)
    w2b = jax.scipy.linalg.block_diag(W2, W2, W2, W2)
    w3b = jax.scipy.linalg.block_diag(*([W3] * 8))
    b1t = jnp.tile(b1, 4).reshape(1, 128)
    b2t = jnp.tile(b2, 8).reshape(1, 128)
    b3t = jnp.tile(b3, 8).reshape(1, 16)
    pswap = jnp.kron(jnp.eye(8, dtype=_f32),
                     jnp.array([[0.0, 1.0], [1.0, 0.0]], _f32))

    dinv1, db32, db16 = _hist(dst_p)
    xw1 = _prep(x_p4, w1b)
    agg1 = _agg32s(xw1.reshape(N_PAD, H1), dinv1, src_p, dst_p)
    xw2 = _mid1(agg1.reshape(NC, P4R, 128), xw1,
                db32.reshape(P4R, 128), b1t, w2b)
    xw2l = xw2.reshape(N_PAD, H2)
    agg2 = _agg16s(xw2l, dinv1, src_p, dst_p)
    xs3 = _mid2(agg2.reshape(NC, P8R, 128), xw2l.reshape(P8R, 128),
                db16.reshape(P8R, 128), b2t)
    agg3 = _agg16n(xs3.reshape(N_PAD, H2), src_p, dst_p)
    ls = _final(agg3.reshape(NC, P8R, 128), xs3, db16.reshape(P8R, 128),
                w3b, b3t, pswap)
    return ls.reshape(N_PAD, D_OUT)[:N]
